# baseline (speedup 1.0000x reference)
import functools

import numpy as np
import jax
import jax.numpy as jnp
from jax import lax
from jax.experimental import pallas as pl
from jax.experimental.pallas import tpu as pltpu
from jax.experimental.pallas import tpu_sc as plsc

F = 26
V = 104000
D = 16
B = 4096
NPAIR = (F * (F - 1)) // 2
RPT = F - 1

NC, NS, L = 2, 16, 16
NW = NC * NS
ROWS_W = B // NW
G = 2
NG = ROWS_W // G
RPTG = RPT * G
TSTRIDE = 56
IDX_PER_G = F * TSTRIDE

_chunk_starts = []
_s = 0
while _s + L <= RPTG:
    _chunk_starts.append(_s)
    _s += L
if _s < RPTG:
    _chunk_starts.append(TSTRIDE - L)
NCH = len(_chunk_starts)
_selt = np.zeros((F, NCH, L), np.int32)
for _t in range(F):
    for _c, _st in enumerate(_chunk_starts):
        for _l in range(L):
            _q = _st + _l
            if _q >= RPTG:
                _selt[_t, _c, _l] = 0
                continue
            _smp, _rank = divmod(_q, RPT)
            _k = _rank if _rank < _t else _rank + 1
            _selt[_t, _c, _l] = _smp * F + _k
SELT = _selt.reshape(-1)

_pstart = np.cumsum([0] + [RPT - i for i in range(F - 1)]).tolist()

_mesh = plsc.VectorSubcoreMesh(core_axis_name="c", subcore_axis_name="s",
                               num_cores=NC, num_subcores=NS)


@functools.partial(
    pl.kernel,
    out_type=jax.ShapeDtypeStruct((B, NPAIR, D), jnp.float32),
    mesh=_mesh,
    scratch_types=[
        pltpu.VMEM((ROWS_W * F,), jnp.int32),
        pltpu.VMEM((F * NCH * L,), jnp.int32),
        pltpu.VMEM((IDX_PER_G,), jnp.int32),
        pltpu.VMEM((IDX_PER_G, D), jnp.float32),
        pltpu.VMEM((G * NPAIR, D), jnp.float32),
        pltpu.SemaphoreType.DMA,
    ],
    compiler_params=pltpu.CompilerParams(needs_layout_passes=False,
                                         use_tc_tiling_on_sc=False),
)
def _ffm_kernel(xf, selt_h, w3, out3,
                xw, selt, idxb, rowsb, outb, gsem):
    wid = lax.axis_index("s") * NC + lax.axis_index("c")
    base_row = wid * ROWS_W
    pltpu.sync_copy(xf.at[pl.ds(base_row * F, ROWS_W * F)], xw)
    pltpu.sync_copy(selt_h, selt)
    lane = lax.broadcasted_iota(jnp.int32, (L,), 0)

    def step(g, carry):
        xoff = g * (G * F)
        for t in range(F):
            for c in range(NCH):
                sv = selt[pl.ds((t * NCH + c) * L, L)] + xoff
                xv = plsc.load_gather(xw, [sv])
                idxb[pl.ds(t * TSTRIDE + _chunk_starts[c], L)] = xv
        cps = []
        for t in range(F):
            cps.append(pltpu.async_copy(
                w3.at[t].at[idxb.at[pl.ds(t * TSTRIDE, TSTRIDE)]],
                rowsb.at[pl.ds(t * TSTRIDE, TSTRIDE)], gsem))
        for cp in cps:
            cp.wait()
        for i in range(F - 1):
            for s in range(G):
                def mul(jj, cr, i=i, s=s):
                    arowv, browv, orowv = cr
                    av = plsc.load_gather(rowsb, [arowv, lane])
                    bv = plsc.load_gather(rowsb, [browv, lane])
                    plsc.store_scatter(outb, [orowv, lane], av * bv)
                    return (arowv + TSTRIDE, browv + 1, orowv + 1)

                lax.fori_loop(
                    0, RPT - i, mul,
                    (jnp.full((L,), (i + 1) * TSTRIDE + s * RPT + i, jnp.int32),
                     jnp.full((L,), i * TSTRIDE + s * RPT + i, jnp.int32),
                     jnp.full((L,), s * NPAIR + _pstart[i], jnp.int32)),
                    unroll=2)
        for s in range(G):
            pltpu.sync_copy(outb.at[pl.ds(s * NPAIR, NPAIR)],
                            out3.at[base_row + g * G + s])
        return carry

    lax.fori_loop(0, NG, step, 0)


def kernel(x, W):
    xf = x.reshape(-1).astype(jnp.int32)
    return _ffm_kernel(xf, jnp.asarray(SELT), W)


# speedup vs baseline: 1.0523x; 1.0523x over previous
import functools

import numpy as np
import jax
import jax.numpy as jnp
from jax import lax
from jax.experimental import pallas as pl
from jax.experimental.pallas import tpu as pltpu
from jax.experimental.pallas import tpu_sc as plsc

F = 26
V = 104000
D = 16
B = 4096
NPAIR = (F * (F - 1)) // 2
ROW_IDX = 2 * NPAIR

NC, NS, L = 2, 16, 16
NW = NC * NS
ROWS_W = B // NW

_mesh = plsc.VectorSubcoreMesh(core_axis_name="c", subcore_axis_name="s",
                               num_cores=NC, num_subcores=NS)

CC = 512
NCT = 203
VMAIN = NCT * CC
CTAIL = V - VMAIN
UNITS = F * NCT
UPW = 166


@functools.partial(
    pl.kernel,
    out_type=jax.ShapeDtypeStruct((F * V * D,), jnp.float32),
    mesh=_mesh,
    scratch_types=[
        pltpu.VMEM((D, CC), jnp.float32),
        pltpu.VMEM((D, CC), jnp.float32),
        pltpu.VMEM((CC * D,), jnp.float32),
        pltpu.VMEM((CC * D,), jnp.float32),
        pltpu.SemaphoreType.DMA,
        pltpu.SemaphoreType.DMA,
        pltpu.SemaphoreType.DMA,
        pltpu.SemaphoreType.DMA,
    ],
    compiler_params=pltpu.CompilerParams(needs_layout_passes=False,
                                         use_tc_tiling_on_sc=True),
)
def _reformat(wt, wtail, wlin, vin0, vin1, vout0, vout1, si0, si1, so0, so1):
    wid = lax.axis_index("s") * NC + lax.axis_index("c")
    lane = lax.broadcasted_iota(jnp.int32, (L,), 0)

    def xpose(ncols):
        def col(b, off):
            rv = plsc.load_gather(vin0, [lane, jnp.full((L,), b,
                                                        jnp.int32)])
            vout0[pl.ds(off, L)] = rv
            return off + D

        lax.fori_loop(0, ncols, col, jnp.int32(0), unroll=8)

    def stepper(k, carry):
        u = wid * UPW + k

        @pl.when(u < UNITS)
        def _():
            t = u // NCT
            c = u % NCT
            pltpu.sync_copy(wt.at[t, :, pl.ds(c * CC, CC)], vin0)
            xpose(CC)
            pltpu.sync_copy(vout0,
                            wlin.at[pl.ds((t * VMAIN + c * CC) * D, CC * D)])

        return carry

    lax.fori_loop(0, UPW, stepper, 0)
    @pl.when(wid < F)
    def _():
        t = wid
        pltpu.sync_copy(wtail.at[pl.ds(t * CTAIL * D, CTAIL * D)],
                        vout0.at[pl.ds(0, CTAIL * D)])
        pltpu.sync_copy(vout0.at[pl.ds(0, CTAIL * D)],
                        wlin.at[pl.ds((F * VMAIN + t * CTAIL) * D, CTAIL * D)])


G = 4
NG = ROWS_W // G
IDX_PER_G = G * ROW_IDX

_pi = np.array([i for i in range(F - 1) for j in range(i + 1, F)], np.int32)
_pj = np.array([j for i in range(F - 1) for j in range(i + 1, F)], np.int32)
_sel = np.concatenate([_pi, _pj])
_tbl = np.concatenate([_pj, _pi]).astype(np.int64)
VMAIN_B = VMAIN
_offm = (_tbl * VMAIN_B).astype(np.int32)
_offt = (F * VMAIN_B + _tbl * (V - VMAIN_B) - VMAIN_B).astype(np.int32)

_starts = []
for _base in (0, NPAIR):
    _s = 0
    while _s + L <= NPAIR:
        _starts.append(_base + _s)
        _s += L
    if _s < NPAIR:
        _starts.append(_base + NPAIR - L)
NCH = len(_starts)
SELC = np.stack([_sel[s:s + L] for s in _starts]).astype(np.int32)
OFFM = np.stack([_offm[s:s + L] for s in _starts]).astype(np.int32)
OFFT = np.stack([_offt[s:s + L] for s in _starts]).astype(np.int32)
DST = list(_starts)

_gchunks = [128] * (IDX_PER_G // 128)
if IDX_PER_G % 128:
    _gchunks.append(IDX_PER_G % 128)


@functools.partial(
    pl.kernel,
    out_type=jax.ShapeDtypeStruct((B * NPAIR * D,), jnp.float32),
    mesh=_mesh,
    scratch_types=[
        pltpu.VMEM((ROWS_W * F,), jnp.int32),
        pltpu.VMEM((NCH * L,), jnp.int32),
        pltpu.VMEM((NCH * L,), jnp.int32),
        pltpu.VMEM((NCH * L,), jnp.int32),
        pltpu.VMEM((IDX_PER_G,), jnp.int32),
        pltpu.VMEM((IDX_PER_G, D), jnp.float32),
        pltpu.VMEM((G * NPAIR * D,), jnp.float32),
        pltpu.SemaphoreType.DMA,
    ],
    compiler_params=pltpu.CompilerParams(needs_layout_passes=False,
                                         use_tc_tiling_on_sc=False),
)
def _ffm_main(xf, selc_h, offm_h, offt_h, flatw, out,
              xw, selc, offm, offt, idxb, rowsb, outb, gsem):
    wid = lax.axis_index("s") * NC + lax.axis_index("c")
    base_row = wid * ROWS_W
    pltpu.sync_copy(xf.at[pl.ds(base_row * F, ROWS_W * F)], xw)
    pltpu.sync_copy(selc_h, selc)
    pltpu.sync_copy(offm_h, offm)
    pltpu.sync_copy(offt_h, offt)
    lane = lax.broadcasted_iota(jnp.int32, (L,), 0)

    def step(g, carry):
        for r in range(G):
            xoff = (g * G + r) * F
            robase = r * ROW_IDX
            for c in range(NCH):
                sv = selc[pl.ds(c * L, L)] + xoff
                xv = plsc.load_gather(xw, [sv])
                idxb[pl.ds(robase + DST[c], L)] = jnp.where(
                    xv < VMAIN_B, xv + offm[pl.ds(c * L, L)],
                    xv + offt[pl.ds(c * L, L)])
        cps = []
        pos = 0
        for n in _gchunks:
            cps.append(pltpu.async_copy(
                flatw.at[idxb.at[pl.ds(pos, n)]],
                rowsb.at[pl.ds(pos, n)], gsem))
            pos += n
        for cp in cps:
            cp.wait()
        for r in range(G):
            def mul(p, cr, r=r):
                arowv, off = cr
                av = plsc.load_gather(rowsb, [arowv, lane])
                bv = plsc.load_gather(rowsb, [arowv + NPAIR, lane])
                outb[pl.ds(off, L)] = av * bv
                return (arowv + 1, off + L)

            lax.fori_loop(
                0, NPAIR, mul,
                (jnp.full((L,), r * ROW_IDX, jnp.int32),
                 jnp.int32(r * NPAIR * D)),
                unroll=4)
        pltpu.sync_copy(
            outb,
            out.at[pl.ds((base_row + g * G) * NPAIR * D, G * NPAIR * D)])
        return carry

    lax.fori_loop(0, NG, step, 0)


def kernel(x, W):
    xf = x.reshape(-1).astype(jnp.int32)
    wt = jnp.transpose(W, (0, 2, 1))
    wtail = W[:, VMAIN:, :].reshape(-1)
    wlin = _reformat(wt, wtail).reshape(F * V, D)
    out = _ffm_main(xf, jnp.asarray(SELC).reshape(-1),
                    jnp.asarray(OFFM).reshape(-1),
                    jnp.asarray(OFFT).reshape(-1), wlin)
    return out.reshape(B, NPAIR, D)


# speedup vs baseline: 1.0755x; 1.0221x over previous
import functools

import numpy as np
import jax
import jax.numpy as jnp
from jax import lax
from jax.experimental import pallas as pl
from jax.experimental.pallas import tpu as pltpu
from jax.experimental.pallas import tpu_sc as plsc

F = 26
V = 104000
D = 16
B = 4096
NPAIR = (F * (F - 1)) // 2
ROW_IDX = 2 * NPAIR

NC, NS, L = 2, 16, 16
NW = NC * NS
ROWS_W = B // NW

_mesh = plsc.VectorSubcoreMesh(core_axis_name="c", subcore_axis_name="s",
                               num_cores=NC, num_subcores=NS)

CC = 1024
NCT = 101
VMAIN = NCT * CC
CTAIL = V - VMAIN
UNITS = F * NCT
UPW = 83


@functools.partial(
    pl.kernel,
    out_type=jax.ShapeDtypeStruct((F * V * D,), jnp.float32),
    mesh=_mesh,
    scratch_types=[
        pltpu.VMEM((D, CC), jnp.float32),
        pltpu.VMEM((D, CC), jnp.float32),
        pltpu.VMEM((CC * D,), jnp.float32),
        pltpu.VMEM((CC * D,), jnp.float32),
        pltpu.SemaphoreType.DMA,
        pltpu.SemaphoreType.DMA,
        pltpu.SemaphoreType.DMA,
        pltpu.SemaphoreType.DMA,
    ],
    compiler_params=pltpu.CompilerParams(needs_layout_passes=False,
                                         use_tc_tiling_on_sc=True),
)
def _reformat(wt, wtail, wlin, vin0, vin1, vout0, vout1, si0, si1, so0, so1):
    wid = lax.axis_index("s") * NC + lax.axis_index("c")
    lane = lax.broadcasted_iota(jnp.int32, (L,), 0)

    def xpose(ncols):
        def col(b, cr):
            off, bvec = cr
            rv = plsc.load_gather(vin0, [lane, bvec])
            vout0[pl.ds(off, L)] = rv
            return (off + D, bvec + 1)

        lax.fori_loop(0, ncols, col,
                      (jnp.int32(0), jnp.zeros((L,), jnp.int32)), unroll=8)

    def stepper(k, carry):
        u = wid * UPW + k

        @pl.when(u < UNITS)
        def _():
            t = u // NCT
            c = u % NCT
            pltpu.sync_copy(wt.at[t, :, pl.ds(c * CC, CC)], vin0)
            xpose(CC)
            pltpu.sync_copy(vout0,
                            wlin.at[pl.ds((t * VMAIN + c * CC) * D, CC * D)])

        return carry

    lax.fori_loop(0, UPW, stepper, 0)
    @pl.when(wid < F)
    def _():
        t = wid
        pltpu.sync_copy(wtail.at[pl.ds(t * CTAIL * D, CTAIL * D)],
                        vout0.at[pl.ds(0, CTAIL * D)])
        pltpu.sync_copy(vout0.at[pl.ds(0, CTAIL * D)],
                        wlin.at[pl.ds((F * VMAIN + t * CTAIL) * D, CTAIL * D)])


G = 4
NG = ROWS_W // G
IDX_PER_G = G * ROW_IDX

_pi = np.array([i for i in range(F - 1) for j in range(i + 1, F)], np.int32)
_pj = np.array([j for i in range(F - 1) for j in range(i + 1, F)], np.int32)
_sel = np.concatenate([_pi, _pj])
_tbl = np.concatenate([_pj, _pi]).astype(np.int64)
VMAIN_B = VMAIN
_offm = (_tbl * VMAIN_B).astype(np.int32)
_offt = (F * VMAIN_B + _tbl * (V - VMAIN_B) - VMAIN_B).astype(np.int32)

_starts = []
for _base in (0, NPAIR):
    _s = 0
    while _s + L <= NPAIR:
        _starts.append(_base + _s)
        _s += L
    if _s < NPAIR:
        _starts.append(_base + NPAIR - L)
NCH = len(_starts)
SELC = np.stack([_sel[s:s + L] for s in _starts]).astype(np.int32)
OFFM = np.stack([_offm[s:s + L] for s in _starts]).astype(np.int32)
OFFT = np.stack([_offt[s:s + L] for s in _starts]).astype(np.int32)
DST = list(_starts)

_gchunks = [128] * (IDX_PER_G // 128)
if IDX_PER_G % 128:
    _gchunks.append(IDX_PER_G % 128)


@functools.partial(
    pl.kernel,
    out_type=jax.ShapeDtypeStruct((B * NPAIR * D,), jnp.float32),
    mesh=_mesh,
    scratch_types=[
        pltpu.VMEM((ROWS_W * F,), jnp.int32),
        pltpu.VMEM((NCH * L,), jnp.int32),
        pltpu.VMEM((NCH * L,), jnp.int32),
        pltpu.VMEM((NCH * L,), jnp.int32),
        pltpu.VMEM((IDX_PER_G,), jnp.int32),
        pltpu.VMEM((IDX_PER_G, D), jnp.float32),
        pltpu.VMEM((G * NPAIR * D,), jnp.float32),
        pltpu.SemaphoreType.DMA,
    ],
    compiler_params=pltpu.CompilerParams(needs_layout_passes=False,
                                         use_tc_tiling_on_sc=False),
)
def _ffm_main(xf, selc_h, offm_h, offt_h, flatw, out,
              xw, selc, offm, offt, idxb, rowsb, outb, gsem):
    wid = lax.axis_index("s") * NC + lax.axis_index("c")
    base_row = wid * ROWS_W
    pltpu.sync_copy(xf.at[pl.ds(base_row * F, ROWS_W * F)], xw)
    pltpu.sync_copy(selc_h, selc)
    pltpu.sync_copy(offm_h, offm)
    pltpu.sync_copy(offt_h, offt)
    lane = lax.broadcasted_iota(jnp.int32, (L,), 0)

    def step(g, carry):
        for r in range(G):
            xoff = (g * G + r) * F
            robase = r * ROW_IDX
            for c in range(NCH):
                sv = selc[pl.ds(c * L, L)] + xoff
                xv = plsc.load_gather(xw, [sv])
                idxb[pl.ds(robase + DST[c], L)] = jnp.where(
                    xv < VMAIN_B, xv + offm[pl.ds(c * L, L)],
                    xv + offt[pl.ds(c * L, L)])
        cps = []
        pos = 0
        for n in _gchunks:
            cps.append(pltpu.async_copy(
                flatw.at[idxb.at[pl.ds(pos, n)]],
                rowsb.at[pl.ds(pos, n)], gsem))
            pos += n
        for cp in cps:
            cp.wait()
        for r in range(G):
            def mul(p, cr, r=r):
                arowv, off = cr
                av = plsc.load_gather(rowsb, [arowv, lane])
                bv = plsc.load_gather(rowsb, [arowv + NPAIR, lane])
                outb[pl.ds(off, L)] = av * bv
                return (arowv + 1, off + L)

            lax.fori_loop(
                0, NPAIR, mul,
                (jnp.full((L,), r * ROW_IDX, jnp.int32),
                 jnp.int32(r * NPAIR * D)),
                unroll=4)
        pltpu.sync_copy(
            outb,
            out.at[pl.ds((base_row + g * G) * NPAIR * D, G * NPAIR * D)])
        return carry

    lax.fori_loop(0, NG, step, 0)


def kernel(x, W):
    xf = x.reshape(-1).astype(jnp.int32)
    wt = jnp.transpose(W, (0, 2, 1))
    wtail = W[:, VMAIN:, :].reshape(-1)
    wlin = _reformat(wt, wtail).reshape(F * V, D)
    out = _ffm_main(xf, jnp.asarray(SELC).reshape(-1),
                    jnp.asarray(OFFM).reshape(-1),
                    jnp.asarray(OFFT).reshape(-1), wlin)
    return out.reshape(B, NPAIR, D)


# speedup vs baseline: 1.0968x; 1.0198x over previous
import functools

import numpy as np
import jax
import jax.numpy as jnp
from jax import lax
from jax.experimental import pallas as pl
from jax.experimental.pallas import tpu as pltpu
from jax.experimental.pallas import tpu_sc as plsc

F = 26
V = 104000
D = 16
B = 4096
NPAIR = (F * (F - 1)) // 2
ROW_IDX = 2 * NPAIR

NC, NS, L = 2, 16, 16
NW = NC * NS
ROWS_W = B // NW
G = 4
NG = ROWS_W // G
IDX_PER_G = G * ROW_IDX

_pi = np.array([i for i in range(F - 1) for j in range(i + 1, F)], np.int32)
_pj = np.array([j for i in range(F - 1) for j in range(i + 1, F)], np.int32)
_sel = np.concatenate([_pi, _pj])
_off = (np.concatenate([_pj, _pi]).astype(np.int64) * V).astype(np.int32)

_starts = []
for _base in (0, NPAIR):
    _s = 0
    while _s + L <= NPAIR:
        _starts.append(_base + _s)
        _s += L
    if _s < NPAIR:
        _starts.append(_base + NPAIR - L)
NCH = len(_starts)
SELC = np.stack([_sel[s:s + L] for s in _starts]).astype(np.int32)
OFFC = np.stack([_off[s:s + L] for s in _starts]).astype(np.int32)
DST = list(_starts)

_gchunks = [128] * (IDX_PER_G // 128)
if IDX_PER_G % 128:
    _gchunks.append(IDX_PER_G % 128)

_mesh = plsc.VectorSubcoreMesh(core_axis_name="c", subcore_axis_name="s",
                               num_cores=NC, num_subcores=NS)


@functools.partial(
    pl.kernel,
    out_type=jax.ShapeDtypeStruct((B * NPAIR * D,), jnp.float32),
    mesh=_mesh,
    scratch_types=[
        pltpu.VMEM((ROWS_W * F,), jnp.int32),
        pltpu.VMEM((NCH * L,), jnp.int32),
        pltpu.VMEM((NCH * L,), jnp.int32),
        pltpu.VMEM((IDX_PER_G,), jnp.int32),
        pltpu.VMEM((IDX_PER_G, D), jnp.float32),
        pltpu.VMEM((G * NPAIR * D,), jnp.float32),
        pltpu.SemaphoreType.DMA,
    ],
    compiler_params=pltpu.CompilerParams(needs_layout_passes=False,
                                         use_tc_tiling_on_sc=False),
)
def _ffm_kernel(xf, selc_h, offc_h, flatw, out,
                xw, selc, offc, idxb, rowsb, outb, gsem):
    wid = lax.axis_index("s") * NC + lax.axis_index("c")
    base_row = wid * ROWS_W
    pltpu.sync_copy(xf.at[pl.ds(base_row * F, ROWS_W * F)], xw)
    pltpu.sync_copy(selc_h, selc)
    pltpu.sync_copy(offc_h, offc)
    lane = lax.broadcasted_iota(jnp.int32, (L,), 0)

    def step(g, carry):
        for r in range(G):
            xoff = (g * G + r) * F
            robase = r * ROW_IDX
            for c in range(NCH):
                sv = selc[pl.ds(c * L, L)] + xoff
                xv = plsc.load_gather(xw, [sv])
                idxb[pl.ds(robase + DST[c], L)] = xv + offc[pl.ds(c * L, L)]
        cps = []
        pos = 0
        for n in _gchunks:
            cps.append(pltpu.async_copy(
                flatw.at[idxb.at[pl.ds(pos, n)]],
                rowsb.at[pl.ds(pos, n)], gsem))
            pos += n
        for cp in cps:
            cp.wait()
        for r in range(G):
            def mul(p, cr, r=r):
                arowv, off = cr
                av = plsc.load_gather(rowsb, [arowv, lane])
                bv = plsc.load_gather(rowsb, [arowv + NPAIR, lane])
                outb[pl.ds(off, L)] = av * bv
                return (arowv + 1, off + L)

            lax.fori_loop(
                0, NPAIR, mul,
                (jnp.full((L,), r * ROW_IDX, jnp.int32),
                 jnp.int32(r * NPAIR * D)),
                unroll=4)
        pltpu.sync_copy(
            outb,
            out.at[pl.ds((base_row + g * G) * NPAIR * D, G * NPAIR * D)])
        return carry

    lax.fori_loop(0, NG, step, 0)


def kernel(x, W):
    xf = x.reshape(-1).astype(jnp.int32)
    flatw = W.reshape(F * V, D)
    out = _ffm_kernel(xf, jnp.asarray(SELC).reshape(-1),
                      jnp.asarray(OFFC).reshape(-1), flatw)
    return out.reshape(B, NPAIR, D)


# speedup vs baseline: 1.1099x; 1.0120x over previous
import functools

import numpy as np
import jax
import jax.numpy as jnp
from jax import lax
from jax.experimental import pallas as pl
from jax.experimental.pallas import tpu as pltpu
from jax.experimental.pallas import tpu_sc as plsc

F = 26
V = 104000
D = 16
B = 4096
NPAIR = (F * (F - 1)) // 2
ROW_IDX = 2 * NPAIR

NC, NS, L = 2, 16, 16
NW = NC * NS
ROWS_W = B // NW
G = 4
NG = ROWS_W // G
SSTRIDE = 656
IDX_PER_G = G * SSTRIDE

_pi = np.array([i for i in range(F - 1) for j in range(i + 1, F)], np.int32)
_pj = np.array([j for i in range(F - 1) for j in range(i + 1, F)], np.int32)
_sel = np.concatenate([_pi, _pj])
_off = (np.concatenate([_pj, _pi]).astype(np.int64) * V).astype(np.int32)

_starts = []
for _base in (0, NPAIR):
    _s = 0
    while _s + L <= NPAIR:
        _starts.append(_base + _s)
        _s += L
    if _s < NPAIR:
        _starts.append(_base + NPAIR - L)
NCH = len(_starts)
SELC = np.stack([_sel[s:s + L] for s in _starts]).astype(np.int32)
OFFC = np.stack([_off[s:s + L] for s in _starts]).astype(np.int32)
DST = list(_starts)

_schunks = [128] * (ROW_IDX // 128)
if ROW_IDX % 128:
    _schunks.append(ROW_IDX % 128)

_mesh = plsc.VectorSubcoreMesh(core_axis_name="c", subcore_axis_name="s",
                               num_cores=NC, num_subcores=NS)


@functools.partial(
    pl.kernel,
    out_type=jax.ShapeDtypeStruct((B * NPAIR * D,), jnp.float32),
    mesh=_mesh,
    scratch_types=[
        pltpu.VMEM((ROWS_W * F,), jnp.int32),
        pltpu.VMEM((NCH * L,), jnp.int32),
        pltpu.VMEM((NCH * L,), jnp.int32),
        pltpu.VMEM((IDX_PER_G,), jnp.int32),
        pltpu.VMEM((IDX_PER_G, D), jnp.float32),
        pltpu.VMEM((G * NPAIR * D,), jnp.float32),
        pltpu.SemaphoreType.DMA,
        pltpu.SemaphoreType.DMA,
        pltpu.SemaphoreType.DMA,
        pltpu.SemaphoreType.DMA,
    ],
    compiler_params=pltpu.CompilerParams(needs_layout_passes=False,
                                         use_tc_tiling_on_sc=False),
)
def _ffm_kernel(xf, selc_h, offc_h, flatw, out,
                xw, selc, offc, idxb, rowsb, outb, s0, s1, s2, s3):
    sems = (s0, s1, s2, s3)
    wid = lax.axis_index("s") * NC + lax.axis_index("c")
    base_row = wid * ROWS_W
    pltpu.sync_copy(xf.at[pl.ds(base_row * F, ROWS_W * F)], xw)
    pltpu.sync_copy(selc_h, selc)
    pltpu.sync_copy(offc_h, offc)
    lane = lax.broadcasted_iota(jnp.int32, (L,), 0)

    def step(g, carry):
        for r in range(G):
            xoff = (g * G + r) * F
            robase = r * SSTRIDE
            for c in range(NCH):
                sv = selc[pl.ds(c * L, L)] + xoff
                idxb[pl.ds(robase + DST[c], L)] = (
                    plsc.load_gather(xw, [sv]) + offc[pl.ds(c * L, L)])
        cps = [[] for _ in range(G)]
        for r in range(G):
            pos = r * SSTRIDE
            for n in _schunks:
                cps[r].append(pltpu.async_copy(
                    flatw.at[idxb.at[pl.ds(pos, n)]],
                    rowsb.at[pl.ds(pos, n)], sems[r]))
                pos += n
        for r in range(G):
            for cp in cps[r]:
                cp.wait()

            def mul(p, cr, r=r):
                arowv, off = cr
                av = plsc.load_gather(rowsb, [arowv, lane])
                bv = plsc.load_gather(rowsb, [arowv + NPAIR, lane])
                outb[pl.ds(off, L)] = av * bv
                return (arowv + 1, off + L)

            lax.fori_loop(
                0, NPAIR, mul,
                (jnp.full((L,), r * SSTRIDE, jnp.int32),
                 jnp.int32(r * NPAIR * D)),
                unroll=4)
        pltpu.sync_copy(
            outb,
            out.at[pl.ds((base_row + g * G) * NPAIR * D, G * NPAIR * D)])
        return carry

    lax.fori_loop(0, NG, step, 0)


def kernel(x, W):
    xf = x.reshape(-1).astype(jnp.int32)
    flatw = W.reshape(F * V, D)
    out = _ffm_kernel(xf, jnp.asarray(SELC).reshape(-1),
                      jnp.asarray(OFFC).reshape(-1), flatw)
    return out.reshape(B, NPAIR, D)
